# baseline (speedup 1.0000x reference)
import functools

import jax
import jax.numpy as jnp
from jax import lax
from jax.experimental import pallas as pl
from jax.experimental.pallas import tpu as pltpu
from jax.experimental.pallas import tpu_sc as plsc

_NUM_CORES = 2
_NUM_SUBCORES = 16
_NUM_WORKERS = _NUM_CORES * _NUM_SUBCORES


@functools.lru_cache(maxsize=None)
def _build(total: int, chunk: int):
    per_w = total // _NUM_WORKERS
    nchunks = per_w // chunk
    assert per_w * _NUM_WORKERS == total and nchunks * chunk == per_w

    mesh = plsc.VectorSubcoreMesh(core_axis_name="c", subcore_axis_name="s")

    @functools.partial(
        pl.kernel,
        mesh=mesh,
        out_type=jax.ShapeDtypeStruct((total,), jnp.float32),
        scratch_types=[
            pltpu.VMEM((chunk,), jnp.int32),
            pltpu.VMEM((chunk,), jnp.int32),
            pltpu.VMEM((chunk,), jnp.float32),
            pltpu.VMEM((chunk,), jnp.float32),
            pltpu.SemaphoreType.DMA,
            pltpu.SemaphoreType.DMA,
            pltpu.SemaphoreType.DMA,
            pltpu.SemaphoreType.DMA,
            pltpu.SemaphoreType.DMA,
            pltpu.SemaphoreType.DMA,
        ],
    )
    def gather_kernel(idx_hbm, w_hbm, out_hbm, iv0, iv1, ov0, ov1,
                      i0, i1, g0, g1, o0, o1):
        idx_v, out_v = (iv0, iv1), (ov0, ov1)
        isem, gsem, osem = (i0, i1), (g0, g1), (o0, o1)
        wid = lax.axis_index("s") * _NUM_CORES + lax.axis_index("c")
        base = wid * per_w

        def start_idx(i):
            b = i % 2
            return pltpu.async_copy(
                idx_hbm.at[pl.ds(base + i * chunk, chunk)], idx_v[b], isem[b])

        idx_h = {}
        out_h = {}
        idx_h[0] = start_idx(0)
        if nchunks > 1:
            idx_h[1] = start_idx(1)
        for i in range(nchunks):
            b = i % 2
            idx_h[i].wait()
            if i >= 2:
                out_h[i - 2].wait()
            pltpu.async_copy(w_hbm.at[idx_v[b]], out_v[b], gsem[b]).wait()
            out_h[i] = pltpu.async_copy(
                out_v[b], out_hbm.at[pl.ds(base + i * chunk, chunk)], osem[b])
            if i + 2 < nchunks:
                idx_h[i + 2] = start_idx(i + 2)
        for i in range(max(0, nchunks - 2), nchunks):
            out_h[i].wait()

    return gather_kernel


def kernel(indexes, weights):
    b, h = indexes.shape
    total = b * h
    flat = indexes.reshape(total)
    out = _build(total, 25600)(flat, weights)
    return out.reshape(b, h)


# speedup vs baseline: 1.6348x; 1.6348x over previous
import functools

import jax
import jax.numpy as jnp
from jax import lax
from jax.experimental import pallas as pl
from jax.experimental.pallas import tpu as pltpu
from jax.experimental.pallas import tpu_sc as plsc

_NUM_CORES = 2
_NUM_SUBCORES = 16
_NUM_WORKERS = _NUM_CORES * _NUM_SUBCORES


@functools.lru_cache(maxsize=None)
def _build(total: int, dim: int, chunk: int):
    per_w = total // _NUM_WORKERS
    nchunks = per_w // chunk
    assert per_w * _NUM_WORKERS == total and nchunks * chunk == per_w
    slice_sz = (dim // _NUM_SUBCORES) & ~7
    last_sz = dim - (_NUM_SUBCORES - 1) * slice_sz

    mesh = plsc.VectorSubcoreMesh(core_axis_name="c", subcore_axis_name="s")

    @functools.partial(
        pl.kernel,
        mesh=mesh,
        out_type=jax.ShapeDtypeStruct((total,), jnp.float32),
        scratch_types=[
            pltpu.VMEM_SHARED((dim,), jnp.float32),
            pltpu.VMEM((chunk,), jnp.int32),
            pltpu.VMEM((chunk,), jnp.int32),
            pltpu.VMEM((chunk,), jnp.float32),
            pltpu.VMEM((chunk,), jnp.float32),
            pltpu.SemaphoreType.DMA,
            pltpu.SemaphoreType.DMA,
            pltpu.SemaphoreType.DMA,
            pltpu.SemaphoreType.DMA,
            pltpu.SemaphoreType.DMA,
            pltpu.SemaphoreType.DMA,
        ],
    )
    def gather_kernel(idx_hbm, w_hbm, out_hbm, shared, iv0, iv1, ov0, ov1,
                      i0, i1, g0, g1, o0, o1):
        idx_v, out_v = (iv0, iv1), (ov0, ov1)
        isem, gsem, osem = (i0, i1), (g0, g1), (o0, o1)
        sid = lax.axis_index("s")
        wid = sid * _NUM_CORES + lax.axis_index("c")
        base = wid * per_w

        def start_idx(i):
            b = i % 2
            return pltpu.async_copy(
                idx_hbm.at[pl.ds(base + i * chunk, chunk)], idx_v[b], isem[b])

        idx_h = {0: start_idx(0)}
        if nchunks > 1:
            idx_h[1] = start_idx(1)

        off = sid * slice_sz
        sub = slice_sz // 4
        last_sub = last_sz // 4
        assert sub % 8 == 0 and last_sub % 8 == 0 and last_sub * 4 == last_sz

        @pl.when(sid < _NUM_SUBCORES - 1)
        def _():
            for k in range(4):
                pltpu.sync_copy(w_hbm.at[pl.ds(off + k * sub, sub)],
                                ov0.at[pl.ds(0, sub)])
                pltpu.sync_copy(ov0.at[pl.ds(0, sub)],
                                shared.at[pl.ds(off + k * sub, sub)])

        @pl.when(sid == _NUM_SUBCORES - 1)
        def _():
            tail = (_NUM_SUBCORES - 1) * slice_sz
            for k in range(4):
                pltpu.sync_copy(w_hbm.at[pl.ds(tail + k * last_sub, last_sub)],
                                ov0.at[pl.ds(0, last_sub)])
                pltpu.sync_copy(ov0.at[pl.ds(0, last_sub)],
                                shared.at[pl.ds(tail + k * last_sub, last_sub)])

        plsc.subcore_barrier()

        out_h = {}
        for i in range(nchunks):
            b = i % 2
            idx_h[i].wait()
            if i >= 2:
                out_h[i - 2].wait()
            pltpu.async_copy(shared.at[idx_v[b]], out_v[b], gsem[b]).wait()
            out_h[i] = pltpu.async_copy(
                out_v[b], out_hbm.at[pl.ds(base + i * chunk, chunk)], osem[b])
            if i + 2 < nchunks:
                idx_h[i + 2] = start_idx(i + 2)
        for i in range(max(0, nchunks - 2), nchunks):
            out_h[i].wait()

    return gather_kernel


def kernel(indexes, weights):
    b, h = indexes.shape
    total = b * h
    flat = indexes.reshape(total)
    out = _build(total, weights.shape[0], 12800)(flat, weights)
    return out.reshape(b, h)


# speedup vs baseline: 2.3214x; 1.4200x over previous
import functools

import jax
import jax.numpy as jnp
from jax import lax
from jax.experimental import pallas as pl
from jax.experimental.pallas import tpu as pltpu
from jax.experimental.pallas import tpu_sc as plsc

_NUM_CORES = 2
_NUM_SUBCORES = 16
_NUM_WORKERS = _NUM_CORES * _NUM_SUBCORES


@functools.lru_cache(maxsize=None)
def _build(ht, bt, dim):
    assert ht % 8 == 0 and bt % (128 * _NUM_WORKERS) == 0
    nrows = ht // 8
    stripe = bt // _NUM_WORKERS
    chunk = 8 * stripe

    slice_sz = (dim // _NUM_SUBCORES) & ~7
    last_sz = dim - (_NUM_SUBCORES - 1) * slice_sz
    bounce = 8192

    mesh = plsc.VectorSubcoreMesh(core_axis_name="c", subcore_axis_name="s")

    @functools.partial(
        pl.kernel,
        mesh=mesh,
        out_type=jax.ShapeDtypeStruct((ht, bt), jnp.float32),
        scratch_types=[
            pltpu.VMEM_SHARED((dim,), jnp.float32),
            pltpu.VMEM((8, 512), jnp.int32),
            pltpu.VMEM((chunk,), jnp.int32),
            pltpu.VMEM((chunk,), jnp.float32),
            pltpu.VMEM((8, 512), jnp.float32),
            pltpu.VMEM((bounce,), jnp.float32),
            pltpu.SemaphoreType.DMA,
        ],
        compiler_params=pltpu.CompilerParams(use_tc_tiling_on_sc=True),
    )
    def gather_kernel(idxT_hbm, w_hbm, outT_hbm, shared, stg_i, idx_v, out_v,
                      stg_o, bnc, gsem):
        sid = lax.axis_index("s")
        wid = sid * _NUM_CORES + lax.axis_index("c")
        col0 = wid * stripe

        def stage(off, sizes):
            o = off
            for sz in sizes:
                pltpu.sync_copy(w_hbm.at[pl.ds(o, sz)], bnc.at[pl.ds(0, sz)])
                pltpu.sync_copy(bnc.at[pl.ds(0, sz)], shared.at[pl.ds(o, sz)])
                o += sz

        def pieces(total):
            full, rem = divmod(total, bounce)
            return [bounce] * full + ([rem] if rem else [])

        @pl.when(sid < _NUM_SUBCORES - 1)
        def _():
            stage(sid * slice_sz, pieces(slice_sz))

        @pl.when(sid == _NUM_SUBCORES - 1)
        def _():
            stage((_NUM_SUBCORES - 1) * slice_sz, pieces(last_sz))

        plsc.subcore_barrier()

        def body(r, _):
            row0 = 8 * r
            pltpu.sync_copy(idxT_hbm.at[pl.ds(row0, 8), pl.ds(col0, stripe)],
                            stg_i)
            for rr in range(8):
                for seg in range(stripe // 16):
                    idx_v[pl.ds(rr * stripe + seg * 16, 16)] = (
                        stg_i[rr, pl.ds(seg * 16, 16)])
            pltpu.async_copy(shared.at[idx_v], out_v, gsem).wait()
            for rr in range(8):
                for seg in range(stripe // 16):
                    stg_o[rr, pl.ds(seg * 16, 16)] = (
                        out_v[pl.ds(rr * stripe + seg * 16, 16)])
            pltpu.sync_copy(stg_o,
                            outT_hbm.at[pl.ds(row0, 8), pl.ds(col0, stripe)])
            return _

        lax.fori_loop(0, nrows, body, None)

    return gather_kernel


def kernel(indexes, weights):
    b, h = indexes.shape
    outT = _build(h, b, weights.shape[0])(indexes.T, weights)
    return outT.T


# speedup vs baseline: 3.4662x; 1.4932x over previous
import functools

import jax
import jax.numpy as jnp
from jax import lax
from jax.experimental import pallas as pl
from jax.experimental.pallas import tpu as pltpu
from jax.experimental.pallas import tpu_sc as plsc

_NUM_CORES = 2
_NUM_SUBCORES = 16
_NUM_WORKERS = _NUM_CORES * _NUM_SUBCORES


@functools.lru_cache(maxsize=None)
def _build(ht, bt, dim):
    assert ht % 8 == 0 and bt % (128 * _NUM_WORKERS) == 0
    nrows = ht // 8
    stripe = bt // _NUM_WORKERS
    chunk = 8 * stripe
    assert nrows % 2 == 1 and nrows >= 5

    slice_sz = (dim // _NUM_SUBCORES) & ~7
    last_sz = dim - (_NUM_SUBCORES - 1) * slice_sz
    bounce = 8192

    mesh = plsc.VectorSubcoreMesh(core_axis_name="c", subcore_axis_name="s")

    @functools.partial(
        pl.kernel,
        mesh=mesh,
        out_type=jax.ShapeDtypeStruct((ht, bt), jnp.float32),
        scratch_types=[
            pltpu.VMEM_SHARED((dim,), jnp.float32),
            pltpu.VMEM((8, 512), jnp.int32),
            pltpu.VMEM((8, 512), jnp.int32),
            pltpu.VMEM((chunk,), jnp.int32),
            pltpu.VMEM((chunk,), jnp.int32),
            pltpu.VMEM((chunk,), jnp.float32),
            pltpu.VMEM((chunk,), jnp.float32),
            pltpu.VMEM((8, 512), jnp.float32),
            pltpu.VMEM((8, 512), jnp.float32),
            pltpu.VMEM((bounce,), jnp.float32),
            pltpu.VMEM((bounce,), jnp.float32),
            pltpu.SemaphoreType.DMA,
            pltpu.SemaphoreType.DMA,
            pltpu.SemaphoreType.DMA,
            pltpu.SemaphoreType.DMA,
            pltpu.SemaphoreType.DMA,
            pltpu.SemaphoreType.DMA,
            pltpu.SemaphoreType.DMA,
            pltpu.SemaphoreType.DMA,
        ],
        compiler_params=pltpu.CompilerParams(use_tc_tiling_on_sc=True),
    )
    def gather_kernel(idxT_hbm, w_hbm, outT_hbm, shared,
                      si0, si1, iv0, iv1, ov0, ov1, so0, so1, bn0, bn1,
                      is0, is1, gs0, gs1, os0, os1, ss0, ss1):
        stg_i, idx_v, out_v, stg_o = (si0, si1), (iv0, iv1), (ov0, ov1), (so0, so1)
        isem, gsem, osem, ssem = (is0, is1), (gs0, gs1), (os0, os1), (ss0, ss1)
        bnc = (bn0, bn1)
        sid = lax.axis_index("s")
        wid = sid * _NUM_CORES + lax.axis_index("c")
        col0 = wid * stripe

        def rows(j):
            return pl.ds(8 * j, 8)

        def mk_a(j, b):
            return pltpu.make_async_copy(
                idxT_hbm.at[rows(j), pl.ds(col0, stripe)], stg_i[b], isem[b])

        def mk_c(b):
            return pltpu.make_async_copy(shared.at[idx_v[b]], out_v[b], gsem[b])

        def mk_e(j, b):
            return pltpu.make_async_copy(
                stg_o[b], outT_hbm.at[rows(j), pl.ds(col0, stripe)], osem[b])

        def bridge_in(b):
            for rr in range(8):
                for seg in range(stripe // 16):
                    idx_v[b][pl.ds(rr * stripe + seg * 16, 16)] = (
                        stg_i[b][rr, pl.ds(seg * 16, 16)])

        def bridge_out(b):
            for rr in range(8):
                for seg in range(stripe // 16):
                    stg_o[b][rr, pl.ds(seg * 16, 16)] = (
                        out_v[b][pl.ds(rr * stripe + seg * 16, 16)])

        mk_a(0, 0).start()
        mk_a(1, 1).start()

        def stage(off, total):
            full, rem = divmod(total, bounce)
            sizes = [bounce] * full + ([rem] if rem else [])
            loads, stores = {}, {}
            for p, sz in enumerate(sizes):
                o = off + p * bounce
                pb = p % 2
                if p >= 2:
                    stores[p - 2].wait()
                loads[p] = pltpu.async_copy(
                    w_hbm.at[pl.ds(o, sz)], bnc[pb].at[pl.ds(0, sz)], ssem[pb])
                loads[p].wait()
                stores[p] = pltpu.async_copy(
                    bnc[pb].at[pl.ds(0, sz)], shared.at[pl.ds(o, sz)], ssem[pb])
            for p in (len(sizes) - 2, len(sizes) - 1):
                if p >= 0:
                    stores[p].wait()

        @pl.when(sid < _NUM_SUBCORES - 1)
        def _():
            stage(sid * slice_sz, slice_sz)

        @pl.when(sid == _NUM_SUBCORES - 1)
        def _():
            stage((_NUM_SUBCORES - 1) * slice_sz, last_sz)

        plsc.subcore_barrier()

        def sub(i, j, b):
            mk_a(j, b).wait()
            bridge_in(b)

            @pl.when(j >= 1)
            def _():
                mk_c(1 - b).wait()

            mk_c(b).start()

            @pl.when(j >= 1)
            def _():
                @pl.when(j >= 3)
                def _():
                    mk_e(j - 3, 1 - b).wait()

                bridge_out(1 - b)
                mk_e(j - 1, 1 - b).start()

            @pl.when(j + 2 <= nrows - 1)
            def _():
                mk_a(j + 2, b).start()

        def body(i, carry):
            sub(i, 2 * i, 0)
            sub(i, 2 * i + 1, 1)
            return carry

        lax.fori_loop(0, nrows // 2, body, None)

        jl = nrows - 1
        mk_a(jl, 0).wait()
        bridge_in(0)
        mk_c(1).wait()
        mk_c(0).start()
        mk_e(jl - 3, 1).wait()
        bridge_out(1)
        mk_e(jl - 1, 1).start()
        mk_c(0).wait()
        mk_e(jl - 2, 0).wait()
        bridge_out(0)
        mk_e(jl, 0).start()
        mk_e(jl - 1, 1).wait()
        mk_e(jl, 0).wait()

    return gather_kernel


def kernel(indexes, weights):
    b, h = indexes.shape
    outT = _build(h, b, weights.shape[0])(indexes.T, weights)
    return outT.T
